# baseline (speedup 1.0000x reference)
import functools

import jax
import jax.numpy as jnp
from jax import lax
from jax.experimental import pallas as pl
from jax.experimental.pallas import tpu as pltpu
from jax.experimental.pallas import tpu_sc as plsc


def _make_sc_broadcast(B, S, D, dtype):
    info = plsc.get_sparse_core_info()
    NC, NS = info.num_cores, info.num_subcores
    NW = NC * NS
    rows_per_w = S // NW
    CH = 64
    n_ch = rows_per_w // CH
    mesh = plsc.VectorSubcoreMesh(core_axis_name="c", subcore_axis_name="s")

    @functools.partial(
        pl.kernel,
        mesh=mesh,
        out_type=jax.ShapeDtypeStruct((B, S, D), dtype),
        scratch_types=[
            pltpu.SemaphoreType.DMA,
        ],
    )
    def k(table_hbm, out_hbm, sem):
        wid = lax.axis_index("s") * NC + lax.axis_index("c")
        base = wid * rows_per_w
        handles = [
            pltpu.async_copy(
                table_hbm.at[pl.ds(base, rows_per_w)],
                out_hbm.at[b, pl.ds(base, rows_per_w)], sem)
            for b in range(B)
        ]
        for h in handles:
            h.wait()

    return k


def kernel(inputs, table):
    B, S = inputs.shape
    V, D = table.shape
    return _make_sc_broadcast(B, S, D, table.dtype)(table)


# speedup vs baseline: 49.9637x; 49.9637x over previous
import functools

import jax
import jax.numpy as jnp
from jax import lax
from jax.experimental import pallas as pl
from jax.experimental.pallas import tpu as pltpu
from jax.experimental.pallas import tpu_sc as plsc


def _make_sc_broadcast(B, S, D, dtype):
    info = plsc.get_sparse_core_info()
    NC, NS = info.num_cores, info.num_subcores
    NW = NC * NS
    rows_per_w = S // NW
    chunks = (8, 24, 64, 64, 64, 32)
    assert sum(chunks) == rows_per_w
    CH_MAX = max(chunks)
    n_ch = len(chunks)
    NBUF = 2
    offs = [sum(chunks[:i]) for i in range(n_ch)]
    mesh = plsc.VectorSubcoreMesh(core_axis_name="c", subcore_axis_name="s")

    @functools.partial(
        pl.kernel,
        mesh=mesh,
        out_type=jax.ShapeDtypeStruct((B, S, D), dtype),
        scratch_types=[
            pltpu.VMEM((CH_MAX, D), dtype),
            pltpu.VMEM((CH_MAX, D), dtype),
            pltpu.SemaphoreType.DMA,
            pltpu.SemaphoreType.DMA,
            pltpu.SemaphoreType.DMA,
            pltpu.SemaphoreType.DMA,
        ],
    )
    def k(table_hbm, out_hbm, buf0, buf1, in0, in1, out0, out1):
        wid = lax.axis_index("s") * NC + lax.axis_index("c")
        base = wid * rows_per_w
        bufs, in_sems, out_sems = (buf0, buf1), (in0, in1), (out0, out1)
        loads, stores = {}, {}

        def start_load(c):
            n = chunks[c]
            lo = base + offs[c]
            loads[c] = pltpu.async_copy(
                table_hbm.at[pl.ds(lo, n)],
                bufs[c % NBUF].at[pl.ds(0, n)], in_sems[c % NBUF])

        def fire_stores(c):
            n = chunks[c]
            lo = base + offs[c]
            stores[c] = [
                pltpu.async_copy(
                    bufs[c % NBUF].at[pl.ds(0, n)],
                    out_hbm.at[b, pl.ds(lo, n)], out_sems[c % NBUF])
                for b in range(B)
            ]

        for c in range(min(NBUF, n_ch)):
            start_load(c)
        for c in range(n_ch):
            if c >= NBUF:
                for h in stores.pop(c - NBUF):
                    h.wait()
                start_load(c)
            loads.pop(c).wait()
            fire_stores(c)
        for c in sorted(stores):
            for h in stores.pop(c):
                h.wait()

    return k


def kernel(inputs, table):
    B, S = inputs.shape
    V, D = table.shape
    return _make_sc_broadcast(B, S, D, table.dtype)(table)
